# baseline (speedup 1.0000x reference)
import functools

import jax
import jax.numpy as jnp
from jax import lax
from jax.experimental import pallas as pl
from jax.experimental.pallas import tpu as pltpu
from jax.experimental.pallas import tpu_sc as plsc

N = 10000
NP = 10240
E = 160000
EP = 160256
D = 256
DH = 128
L = 16
NS = 16
NC = 2
CH = 32
RING = 4
NCH = EP // (NS * CH)
EPT = EP // NS
SL = NP // NS
RB = 1024
NRB = NP // RB
DUMP = N

_mesh = plsc.VectorSubcoreMesh(core_axis_name="c", subcore_axis_name="s")


@functools.partial(
    pl.kernel,
    out_type=jax.ShapeDtypeStruct((NC, NP), jnp.float32),
    mesh=_mesh,
    scratch_types=[
        pltpu.VMEM((EPT,), jnp.int32),
        pltpu.VMEM((NP,), jnp.float32),
        pltpu.VMEM((SL,), jnp.float32),
        pltpu.VMEM((SL,), jnp.float32),
        pltpu.VMEM_SHARED((NS, NP), jnp.float32),
    ],
    compiler_params=pltpu.CompilerParams(needs_layout_passes=False),
)
def _sc_degrees(idx_hbm, out_hbm, idx_v, deg_v, acc_v, tmp_v, part_sh):
    c = lax.axis_index("c")
    s = lax.axis_index("s")
    pltpu.sync_copy(idx_hbm.at[c].at[s], idx_v)
    zeros = jnp.zeros((L,), jnp.float32)
    ones = jnp.ones((L,), jnp.float32)

    def zero_deg(i, _):
        deg_v[pl.ds(i * L, L)] = zeros
        return 0

    lax.fori_loop(0, NP // L, zero_deg, 0)

    def accum(i, _):
        iv = idx_v[pl.ds(i * L, L)]
        plsc.addupdate_scatter(deg_v, [iv], ones)
        return 0

    lax.fori_loop(0, EPT // L, accum, 0)
    pltpu.sync_copy(deg_v, part_sh.at[s])
    plsc.subcore_barrier()

    def zero_acc(i, _):
        acc_v[pl.ds(i * L, L)] = zeros
        return 0

    lax.fori_loop(0, SL // L, zero_acc, 0)

    def reduce_tile(t, _):
        pltpu.sync_copy(part_sh.at[t].at[pl.ds(s * SL, SL)], tmp_v)

        def addv(i, _):
            acc_v[pl.ds(i * L, L)] = acc_v[pl.ds(i * L, L)] + tmp_v[pl.ds(i * L, L)]
            return 0

        lax.fori_loop(0, SL // L, addv, 0)
        return 0

    lax.fori_loop(0, NS, reduce_tile, 0)
    pltpu.sync_copy(acc_v, out_hbm.at[c].at[pl.ds(s * SL, SL)])


@functools.partial(
    pl.kernel,
    out_type=jax.ShapeDtypeStruct((NC, NP, DH), jnp.float32),
    mesh=_mesh,
    scratch_types=[
        pltpu.VMEM((RING, 2, CH), jnp.int32),
        pltpu.VMEM((RING, CH, DH), jnp.float32),
        pltpu.VMEM_SHARED((NP, DH), jnp.float32),
        pltpu.SemaphoreType.DMA((RING,)),
        pltpu.SemaphoreType.DMA((RING,)),
    ],
)
def _sc_message(h_hbm, ip_hbm, out_hbm, pair_v, rows_v, agg_sh, rsems, isems):
    c = lax.axis_index("c")
    s = lax.axis_index("s")
    zeros = jnp.zeros((L,), jnp.float32)

    def zero_rows(i, _):
        rows_v[0, i // (DH // L), pl.ds((i % (DH // L)) * L, L)] = zeros
        return 0

    lax.fori_loop(0, CH * DH // L, zero_rows, 0)
    for t in range(SL // CH):
        pltpu.sync_copy(rows_v.at[0], agg_sh.at[pl.ds(s * SL + t * CH, CH)])

    def idx_fetch(j, slot):
        return pltpu.async_copy(
            ip_hbm.at[c].at[s].at[j], pair_v.at[slot], isems.at[slot]
        )

    def gather(j, slot):
        pltpu.async_copy(
            h_hbm.at[pair_v.at[slot].at[0]], rows_v.at[slot], rsems.at[slot]
        )

    for k in range(RING - 1):
        idx_fetch(k, k).wait()
        gather(k, k)
    idx_fetch(RING - 1, RING - 1)
    plsc.subcore_barrier()

    def body(j, _):
        b = lax.rem(j, RING)
        pltpu.make_async_copy(
            h_hbm.at[pair_v.at[b].at[0]], rows_v.at[b], rsems.at[b]
        ).wait()

        @pl.when(j < NCH - (RING - 1))
        def _next_gather():
            slot2 = lax.rem(j + RING - 1, RING)
            pltpu.make_async_copy(
                ip_hbm.at[c].at[s].at[j + RING - 1],
                pair_v.at[slot2],
                isems.at[slot2],
            ).wait()
            gather(j + RING - 1, slot2)

        pltpu.sync_copy(rows_v.at[b], agg_sh.at[pair_v.at[b].at[1]], add=True)

        @pl.when(j < NCH - RING)
        def _next_idx():
            idx_fetch(j + RING, b)

        return 0

    lax.fori_loop(0, NCH, body, 0)
    plsc.subcore_barrier()
    pltpu.sync_copy(
        agg_sh.at[pl.ds(s * SL, SL)], out_hbm.at[c].at[pl.ds(s * SL, SL)]
    )


def _rsqrt_deg(deg_blk):
    return lax.rsqrt(jnp.maximum(deg_blk.reshape(RB // DH, DH), 1.0))


def _tc_h1(xp, W1, deg3):
    def body(x_ref, w_ref, deg_ref, out_ref):
        outs = _rsqrt_deg(deg_ref[...])
        acc = jnp.dot(x_ref[...], w_ref[...], preferred_element_type=jnp.float32)
        acc = acc.reshape(RB // DH, DH, DH) * outs[:, :, None]
        out_ref[...] = acc.reshape(1, RB, DH)

    return pl.pallas_call(
        body,
        grid=(NRB, NC),
        in_specs=[
            pl.BlockSpec((RB, D), lambda r, c: (r, 0)),
            pl.BlockSpec((D, DH), lambda r, c: (0, c)),
            pl.BlockSpec((1, RB // DH, DH), lambda r, c: (0, r, 0)),
        ],
        out_specs=pl.BlockSpec((1, RB, DH), lambda r, c: (c, r, 0)),
        out_shape=jax.ShapeDtypeStruct((NC, NP, DH), jnp.float32),
    )(xp, W1, deg3)


def _tc_h2(agg1, W2, deg3, b2d):
    def body(a0_ref, a1_ref, w0_ref, w1_ref, din_ref, dout_ref, b_ref, out_ref):
        ins = _rsqrt_deg(din_ref[...])
        outs = _rsqrt_deg(dout_ref[...])

        def mk(a_ref, bh):
            a = a_ref[...].reshape(RB // DH, DH, DH)
            m = jnp.maximum(a * ins[:, :, None] + bh, 0.0) * outs[:, :, None]
            return m.reshape(RB, DH)

        m0 = mk(a0_ref, b_ref[0, :])
        m1 = mk(a1_ref, b_ref[1, :])
        out = jnp.dot(m0, w0_ref[...], preferred_element_type=jnp.float32)
        out = out + jnp.dot(m1, w1_ref[...], preferred_element_type=jnp.float32)
        out_ref[...] = out.reshape(1, RB, DH)

    return pl.pallas_call(
        body,
        grid=(NRB, NC),
        in_specs=[
            pl.BlockSpec((1, RB, DH), lambda r, c: (0, r, 0)),
            pl.BlockSpec((1, RB, DH), lambda r, c: (1, r, 0)),
            pl.BlockSpec((DH, DH), lambda r, c: (0, c)),
            pl.BlockSpec((DH, DH), lambda r, c: (1, c)),
            pl.BlockSpec((1, RB // DH, DH), lambda r, c: (1, r, 0)),
            pl.BlockSpec((1, RB // DH, DH), lambda r, c: (0, r, 0)),
            pl.BlockSpec((2, DH), lambda r, c: (0, 0)),
        ],
        out_specs=pl.BlockSpec((1, RB, DH), lambda r, c: (c, r, 0)),
        out_shape=jax.ShapeDtypeStruct((NC, NP, DH), jnp.float32),
    )(agg1, agg1, W2, W2, deg3, deg3, b2d)


def _tc_final(agg2, deg3, b2d):
    def body(a0_ref, a1_ref, din_ref, b_ref, h_ref, pool_ref, mx_sc, sm_sc):
        r = pl.program_id(0)
        ins = _rsqrt_deg(din_ref[...])

        def mk(a_ref, bh):
            a = a_ref[...].reshape(RB // DH, DH, DH)
            return (a * ins[:, :, None] + bh).reshape(RB, DH)

        hb = jnp.concatenate([mk(a0_ref, b_ref[0, :]), mk(a1_ref, b_ref[1, :])], axis=1)
        h_ref[...] = hb
        row = r * RB + lax.broadcasted_iota(jnp.int32, (RB, 1), 0)
        valid = row < N
        bmax = jnp.max(jnp.where(valid, hb, -jnp.inf), axis=0, keepdims=True)
        bsum = jnp.sum(jnp.where(valid, hb, 0.0), axis=0, keepdims=True)

        @pl.when(r == 0)
        def _init():
            mx_sc[...] = bmax
            sm_sc[...] = bsum

        @pl.when(r > 0)
        def _acc():
            mx_sc[...] = jnp.maximum(mx_sc[...], bmax)
            sm_sc[...] = sm_sc[...] + bsum

        @pl.when(r == NRB - 1)
        def _fin():
            pool_ref[...] = mx_sc[...] + sm_sc[...] * (1.0 / N)

    return pl.pallas_call(
        body,
        grid=(NRB,),
        in_specs=[
            pl.BlockSpec((1, RB, DH), lambda r: (0, r, 0)),
            pl.BlockSpec((1, RB, DH), lambda r: (1, r, 0)),
            pl.BlockSpec((1, RB // DH, DH), lambda r: (1, r, 0)),
            pl.BlockSpec((2, DH), lambda r: (0, 0)),
        ],
        out_specs=[
            pl.BlockSpec((RB, D), lambda r: (r, 0)),
            pl.BlockSpec((1, D), lambda r: (0, 0)),
        ],
        out_shape=[
            jax.ShapeDtypeStruct((NP, D), jnp.float32),
            jax.ShapeDtypeStruct((1, D), jnp.float32),
        ],
        scratch_shapes=[
            pltpu.VMEM((1, D), jnp.float32),
            pltpu.VMEM((1, D), jnp.float32),
        ],
    )(agg2, agg2, deg3, b2d)


def kernel(x, edge_index, W1, b1, W2, b2):
    src = edge_index[0].astype(jnp.int32)
    dst = edge_index[1].astype(jnp.int32)
    pad = jnp.full((EP - E,), DUMP, jnp.int32)
    srcp = jnp.concatenate([src, pad])
    dstp = jnp.concatenate([dst, pad])
    idx_deg = jnp.stack([srcp, dstp]).reshape(2, NS, EPT)
    src3 = srcp.reshape(NS, NCH, CH)
    dst3 = dstp.reshape(NS, NCH, CH)
    idx_pair = jnp.stack(
        [
            jnp.stack([src3, dst3], axis=2),
            jnp.stack([src3 + NP, dst3], axis=2),
        ]
    )
    xp = jnp.pad(x, ((0, NP - N), (0, 0)))

    deg = _sc_degrees(idx_deg)
    deg3 = deg.reshape(2, NP // DH, DH)
    h1 = _tc_h1(xp, W1, deg3)
    agg1 = _sc_message(h1.reshape(NC * NP, DH), idx_pair)
    h2 = _tc_h2(agg1, W2, deg3, b1.reshape(2, DH))
    agg2 = _sc_message(h2.reshape(NC * NP, DH), idx_pair)
    hp, pool_x = _tc_final(agg2, deg3, b2.reshape(2, DH))
    return (pool_x, hp[:N])


# speedup vs baseline: 1.2744x; 1.2744x over previous
import functools

import jax
import jax.numpy as jnp
from jax import lax
from jax.experimental import pallas as pl
from jax.experimental.pallas import tpu as pltpu
from jax.experimental.pallas import tpu_sc as plsc

N = 10000
NP = 10240
E = 160000
EP = 160768
D = 256
DH = 128
L = 16
NS = 16
NC = 2
CH = 64
RING = 5
NCH = EP // (NS * CH)
EPT = EP // NS
SL = NP // NS
RB = 1024
NRB = NP // RB
DUMP = N

_mesh = plsc.VectorSubcoreMesh(core_axis_name="c", subcore_axis_name="s")


@functools.partial(
    pl.kernel,
    out_type=jax.ShapeDtypeStruct((NC, NP), jnp.float32),
    mesh=_mesh,
    scratch_types=[
        pltpu.VMEM((EPT,), jnp.int32),
        pltpu.VMEM((NP,), jnp.float32),
        pltpu.VMEM((SL,), jnp.float32),
        pltpu.VMEM((SL,), jnp.float32),
        pltpu.VMEM_SHARED((NS, NP), jnp.float32),
    ],
    compiler_params=pltpu.CompilerParams(needs_layout_passes=False),
)
def _sc_degrees(idx_hbm, out_hbm, idx_v, deg_v, acc_v, tmp_v, part_sh):
    c = lax.axis_index("c")
    s = lax.axis_index("s")
    pltpu.sync_copy(idx_hbm.at[c].at[s], idx_v)
    zeros = jnp.zeros((L,), jnp.float32)
    ones = jnp.ones((L,), jnp.float32)

    def zero_deg(i, _):
        deg_v[pl.ds(i * L, L)] = zeros
        return 0

    lax.fori_loop(0, NP // L, zero_deg, 0)

    def accum(i, _):
        iv = idx_v[pl.ds(i * L, L)]
        plsc.addupdate_scatter(deg_v, [iv], ones)
        return 0

    lax.fori_loop(0, EPT // L, accum, 0)
    pltpu.sync_copy(deg_v, part_sh.at[s])
    plsc.subcore_barrier()

    def zero_acc(i, _):
        acc_v[pl.ds(i * L, L)] = zeros
        return 0

    lax.fori_loop(0, SL // L, zero_acc, 0)

    def reduce_tile(t, _):
        pltpu.sync_copy(part_sh.at[t].at[pl.ds(s * SL, SL)], tmp_v)

        def addv(i, _):
            acc_v[pl.ds(i * L, L)] = acc_v[pl.ds(i * L, L)] + tmp_v[pl.ds(i * L, L)]
            return 0

        lax.fori_loop(0, SL // L, addv, 0)
        return 0

    lax.fori_loop(0, NS, reduce_tile, 0)
    pltpu.sync_copy(acc_v, out_hbm.at[c].at[pl.ds(s * SL, SL)])


@functools.partial(
    pl.kernel,
    out_type=jax.ShapeDtypeStruct((NC, NP, DH), jnp.float32),
    mesh=_mesh,
    scratch_types=[
        pltpu.VMEM((RING, 2, CH), jnp.int32),
        pltpu.VMEM((RING, CH, DH), jnp.float32),
        pltpu.VMEM_SHARED((NP, DH), jnp.float32),
        pltpu.SemaphoreType.DMA((RING,)),
        pltpu.SemaphoreType.DMA((RING,)),
    ],
)
def _sc_message(h_hbm, ip_hbm, out_hbm, pair_v, rows_v, agg_sh, rsems, isems):
    c = lax.axis_index("c")
    s = lax.axis_index("s")
    zeros = jnp.zeros((L,), jnp.float32)

    def zero_rows(i, _):
        rows_v[0, i // (DH // L), pl.ds((i % (DH // L)) * L, L)] = zeros
        return 0

    lax.fori_loop(0, CH * DH // L, zero_rows, 0)
    for t in range(SL // CH):
        pltpu.sync_copy(rows_v.at[0], agg_sh.at[pl.ds(s * SL + t * CH, CH)])

    def idx_fetch(j, slot):
        return pltpu.async_copy(
            ip_hbm.at[c].at[s].at[j], pair_v.at[slot], isems.at[slot]
        )

    def gather(j, slot):
        pltpu.async_copy(
            h_hbm.at[pair_v.at[slot].at[0]], rows_v.at[slot], rsems.at[slot]
        )

    for k in range(RING - 1):
        idx_fetch(k, k).wait()
        gather(k, k)
    idx_fetch(RING - 1, RING - 1)
    plsc.subcore_barrier()

    def body(j, _):
        b = lax.rem(j, RING)
        pltpu.make_async_copy(
            h_hbm.at[pair_v.at[b].at[0]], rows_v.at[b], rsems.at[b]
        ).wait()

        @pl.when(j < NCH - (RING - 1))
        def _next_gather():
            slot2 = lax.rem(j + RING - 1, RING)
            pltpu.make_async_copy(
                ip_hbm.at[c].at[s].at[j + RING - 1],
                pair_v.at[slot2],
                isems.at[slot2],
            ).wait()
            gather(j + RING - 1, slot2)

        pltpu.sync_copy(rows_v.at[b], agg_sh.at[pair_v.at[b].at[1]], add=True)

        @pl.when(j < NCH - RING)
        def _next_idx():
            idx_fetch(j + RING, b)

        return 0

    lax.fori_loop(0, NCH, body, 0)
    plsc.subcore_barrier()
    pltpu.sync_copy(
        agg_sh.at[pl.ds(s * SL, SL)], out_hbm.at[c].at[pl.ds(s * SL, SL)]
    )


def _rsqrt_deg(deg_blk):
    return lax.rsqrt(jnp.maximum(deg_blk.reshape(RB // DH, DH), 1.0))


def _tc_h1(xp, W1, deg3):
    def body(x_ref, w_ref, deg_ref, out_ref):
        outs = _rsqrt_deg(deg_ref[...])
        acc = jnp.dot(x_ref[...], w_ref[...], preferred_element_type=jnp.float32)
        acc = acc.reshape(RB // DH, DH, DH) * outs[:, :, None]
        out_ref[...] = acc.reshape(1, RB, DH)

    return pl.pallas_call(
        body,
        grid=(NRB, NC),
        in_specs=[
            pl.BlockSpec((RB, D), lambda r, c: (r, 0)),
            pl.BlockSpec((D, DH), lambda r, c: (0, c)),
            pl.BlockSpec((1, RB // DH, DH), lambda r, c: (0, r, 0)),
        ],
        out_specs=pl.BlockSpec((1, RB, DH), lambda r, c: (c, r, 0)),
        out_shape=jax.ShapeDtypeStruct((NC, NP, DH), jnp.float32),
    )(xp, W1, deg3)


def _tc_h2(agg1, W2, deg3, b2d):
    def body(a0_ref, a1_ref, w0_ref, w1_ref, din_ref, dout_ref, b_ref, out_ref):
        ins = _rsqrt_deg(din_ref[...])
        outs = _rsqrt_deg(dout_ref[...])

        def mk(a_ref, bh):
            a = a_ref[...].reshape(RB // DH, DH, DH)
            m = jnp.maximum(a * ins[:, :, None] + bh, 0.0) * outs[:, :, None]
            return m.reshape(RB, DH)

        m0 = mk(a0_ref, b_ref[0, :])
        m1 = mk(a1_ref, b_ref[1, :])
        out = jnp.dot(m0, w0_ref[...], preferred_element_type=jnp.float32)
        out = out + jnp.dot(m1, w1_ref[...], preferred_element_type=jnp.float32)
        out_ref[...] = out.reshape(1, RB, DH)

    return pl.pallas_call(
        body,
        grid=(NRB, NC),
        in_specs=[
            pl.BlockSpec((1, RB, DH), lambda r, c: (0, r, 0)),
            pl.BlockSpec((1, RB, DH), lambda r, c: (1, r, 0)),
            pl.BlockSpec((DH, DH), lambda r, c: (0, c)),
            pl.BlockSpec((DH, DH), lambda r, c: (1, c)),
            pl.BlockSpec((1, RB // DH, DH), lambda r, c: (1, r, 0)),
            pl.BlockSpec((1, RB // DH, DH), lambda r, c: (0, r, 0)),
            pl.BlockSpec((2, DH), lambda r, c: (0, 0)),
        ],
        out_specs=pl.BlockSpec((1, RB, DH), lambda r, c: (c, r, 0)),
        out_shape=jax.ShapeDtypeStruct((NC, NP, DH), jnp.float32),
    )(agg1, agg1, W2, W2, deg3, deg3, b2d)


def _tc_final(agg2, deg3, b2d):
    def body(a0_ref, a1_ref, din_ref, b_ref, h_ref, pool_ref, mx_sc, sm_sc):
        r = pl.program_id(0)
        ins = _rsqrt_deg(din_ref[...])

        def mk(a_ref, bh):
            a = a_ref[...].reshape(RB // DH, DH, DH)
            return (a * ins[:, :, None] + bh).reshape(RB, DH)

        hb = jnp.concatenate([mk(a0_ref, b_ref[0, :]), mk(a1_ref, b_ref[1, :])], axis=1)
        h_ref[...] = hb
        row = r * RB + lax.broadcasted_iota(jnp.int32, (RB, 1), 0)
        valid = row < N
        bmax = jnp.max(jnp.where(valid, hb, -jnp.inf), axis=0, keepdims=True)
        bsum = jnp.sum(jnp.where(valid, hb, 0.0), axis=0, keepdims=True)

        @pl.when(r == 0)
        def _init():
            mx_sc[...] = bmax
            sm_sc[...] = bsum

        @pl.when(r > 0)
        def _acc():
            mx_sc[...] = jnp.maximum(mx_sc[...], bmax)
            sm_sc[...] = sm_sc[...] + bsum

        @pl.when(r == NRB - 1)
        def _fin():
            pool_ref[...] = mx_sc[...] + sm_sc[...] * (1.0 / N)

    return pl.pallas_call(
        body,
        grid=(NRB,),
        in_specs=[
            pl.BlockSpec((1, RB, DH), lambda r: (0, r, 0)),
            pl.BlockSpec((1, RB, DH), lambda r: (1, r, 0)),
            pl.BlockSpec((1, RB // DH, DH), lambda r: (1, r, 0)),
            pl.BlockSpec((2, DH), lambda r: (0, 0)),
        ],
        out_specs=[
            pl.BlockSpec((RB, D), lambda r: (r, 0)),
            pl.BlockSpec((1, D), lambda r: (0, 0)),
        ],
        out_shape=[
            jax.ShapeDtypeStruct((NP, D), jnp.float32),
            jax.ShapeDtypeStruct((1, D), jnp.float32),
        ],
        scratch_shapes=[
            pltpu.VMEM((1, D), jnp.float32),
            pltpu.VMEM((1, D), jnp.float32),
        ],
    )(agg2, agg2, deg3, b2d)


def kernel(x, edge_index, W1, b1, W2, b2):
    src = edge_index[0].astype(jnp.int32)
    dst = edge_index[1].astype(jnp.int32)
    pad = jnp.full((EP - E,), DUMP, jnp.int32)
    srcp = jnp.concatenate([src, pad])
    dstp = jnp.concatenate([dst, pad])
    idx_deg = jnp.stack([srcp, dstp]).reshape(2, NS, EPT)
    src3 = srcp.reshape(NS, NCH, CH)
    dst3 = dstp.reshape(NS, NCH, CH)
    idx_pair = jnp.stack(
        [
            jnp.stack([src3, dst3], axis=2),
            jnp.stack([src3 + NP, dst3], axis=2),
        ]
    )
    xp = jnp.pad(x, ((0, NP - N), (0, 0)))

    deg = _sc_degrees(idx_deg)
    deg3 = deg.reshape(2, NP // DH, DH)
    h1 = _tc_h1(xp, W1, deg3)
    agg1 = _sc_message(h1.reshape(NC * NP, DH), idx_pair)
    h2 = _tc_h2(agg1, W2, deg3, b1.reshape(2, DH))
    agg2 = _sc_message(h2.reshape(NC * NP, DH), idx_pair)
    hp, pool_x = _tc_final(agg2, deg3, b2.reshape(2, DH))
    return (pool_x, hp[:N])


# speedup vs baseline: 1.2746x; 1.0002x over previous
import functools

import jax
import jax.numpy as jnp
from jax import lax
from jax.experimental import pallas as pl
from jax.experimental.pallas import tpu as pltpu
from jax.experimental.pallas import tpu_sc as plsc

N = 10000
NP = 10240
E = 160000
EP = 160768
D = 256
DH = 128
L = 16
NS = 16
NC = 2
CH = 64
RING = 4
NCH = EP // (NS * CH)
EPT = EP // NS
SL = NP // NS
RB = 1024
NRB = NP // RB
DUMP = N

_mesh = plsc.VectorSubcoreMesh(core_axis_name="c", subcore_axis_name="s")


@functools.partial(
    pl.kernel,
    out_type=jax.ShapeDtypeStruct((NC, NP), jnp.float32),
    mesh=_mesh,
    scratch_types=[
        pltpu.VMEM((EPT,), jnp.int32),
        pltpu.VMEM((NP,), jnp.float32),
        pltpu.VMEM((SL,), jnp.float32),
        pltpu.VMEM((SL,), jnp.float32),
        pltpu.VMEM_SHARED((NS, NP), jnp.float32),
    ],
    compiler_params=pltpu.CompilerParams(needs_layout_passes=False),
)
def _sc_degrees(idx_hbm, out_hbm, idx_v, deg_v, acc_v, tmp_v, part_sh):
    c = lax.axis_index("c")
    s = lax.axis_index("s")
    pltpu.sync_copy(idx_hbm.at[c].at[s], idx_v)
    zeros = jnp.zeros((L,), jnp.float32)
    ones = jnp.ones((L,), jnp.float32)

    def zero_deg(i, _):
        deg_v[pl.ds(i * L, L)] = zeros
        return 0

    lax.fori_loop(0, NP // L, zero_deg, 0)

    def accum(i, _):
        iv = idx_v[pl.ds(i * L, L)]
        plsc.addupdate_scatter(deg_v, [iv], ones)
        return 0

    lax.fori_loop(0, EPT // L, accum, 0)
    pltpu.sync_copy(deg_v, part_sh.at[s])
    plsc.subcore_barrier()

    def zero_acc(i, _):
        acc_v[pl.ds(i * L, L)] = zeros
        return 0

    lax.fori_loop(0, SL // L, zero_acc, 0)

    def reduce_tile(t, _):
        pltpu.sync_copy(part_sh.at[t].at[pl.ds(s * SL, SL)], tmp_v)

        def addv(i, _):
            acc_v[pl.ds(i * L, L)] = acc_v[pl.ds(i * L, L)] + tmp_v[pl.ds(i * L, L)]
            return 0

        lax.fori_loop(0, SL // L, addv, 0)
        return 0

    lax.fori_loop(0, NS, reduce_tile, 0)
    pltpu.sync_copy(acc_v, out_hbm.at[c].at[pl.ds(s * SL, SL)])


@functools.partial(
    pl.kernel,
    out_type=jax.ShapeDtypeStruct((NC, NP, DH), jnp.float32),
    mesh=_mesh,
    scratch_types=[
        pltpu.VMEM((RING, 2, CH), jnp.int32),
        pltpu.VMEM((RING, CH, DH), jnp.float32),
        pltpu.VMEM_SHARED((NP, DH), jnp.float32),
        pltpu.SemaphoreType.DMA((RING,)),
        pltpu.SemaphoreType.DMA((RING,)),
    ],
)
def _sc_message(h_hbm, ip_hbm, out_hbm, pair_v, rows_v, agg_sh, rsems, isems):
    c = lax.axis_index("c")
    s = lax.axis_index("s")
    zeros = jnp.zeros((L,), jnp.float32)

    def zero_rows(i, _):
        rows_v[0, i // (DH // L), pl.ds((i % (DH // L)) * L, L)] = zeros
        return 0

    lax.fori_loop(0, CH * DH // L, zero_rows, 0)
    for t in range(SL // CH):
        pltpu.sync_copy(rows_v.at[0], agg_sh.at[pl.ds(s * SL + t * CH, CH)])

    def idx_fetch(j, slot):
        return pltpu.async_copy(
            ip_hbm.at[c].at[s].at[j], pair_v.at[slot], isems.at[slot]
        )

    def gather(j, slot):
        pltpu.async_copy(
            h_hbm.at[pair_v.at[slot].at[0]], rows_v.at[slot], rsems.at[slot]
        )

    for k in range(RING - 1):
        idx_fetch(k, k).wait()
        gather(k, k)
    idx_fetch(RING - 1, RING - 1)
    plsc.subcore_barrier()

    def body(j, _):
        b = lax.rem(j, RING)
        pltpu.make_async_copy(
            h_hbm.at[pair_v.at[b].at[0]], rows_v.at[b], rsems.at[b]
        ).wait()

        @pl.when(j < NCH - (RING - 1))
        def _next_gather():
            slot2 = lax.rem(j + RING - 1, RING)
            pltpu.make_async_copy(
                ip_hbm.at[c].at[s].at[j + RING - 1],
                pair_v.at[slot2],
                isems.at[slot2],
            ).wait()
            gather(j + RING - 1, slot2)

        pltpu.sync_copy(rows_v.at[b], agg_sh.at[pair_v.at[b].at[1]], add=True)

        @pl.when(j < NCH - RING)
        def _next_idx():
            idx_fetch(j + RING, b)

        return 0

    lax.fori_loop(0, NCH, body, 0)
    plsc.subcore_barrier()
    pltpu.sync_copy(
        agg_sh.at[pl.ds(s * SL, SL)], out_hbm.at[c].at[pl.ds(s * SL, SL)]
    )


def _rsqrt_deg(deg_blk):
    return lax.rsqrt(jnp.maximum(deg_blk.reshape(RB // DH, DH), 1.0))


def _tc_h1(xp, W1, deg3):
    def body(x_ref, w_ref, deg_ref, out_ref):
        outs = _rsqrt_deg(deg_ref[...])
        acc = jnp.dot(x_ref[...], w_ref[...], preferred_element_type=jnp.float32)
        acc = acc.reshape(RB // DH, DH, DH) * outs[:, :, None]
        out_ref[...] = acc.reshape(1, RB, DH)

    return pl.pallas_call(
        body,
        grid=(NRB, NC),
        in_specs=[
            pl.BlockSpec((RB, D), lambda r, c: (r, 0)),
            pl.BlockSpec((D, DH), lambda r, c: (0, c)),
            pl.BlockSpec((1, RB // DH, DH), lambda r, c: (0, r, 0)),
        ],
        out_specs=pl.BlockSpec((1, RB, DH), lambda r, c: (c, r, 0)),
        out_shape=jax.ShapeDtypeStruct((NC, NP, DH), jnp.float32),
    )(xp, W1, deg3)


def _tc_h2(agg1, W2, deg3, b2d):
    def body(a0_ref, a1_ref, w0_ref, w1_ref, din_ref, dout_ref, b_ref, out_ref):
        ins = _rsqrt_deg(din_ref[...])
        outs = _rsqrt_deg(dout_ref[...])

        def mk(a_ref, bh):
            a = a_ref[...].reshape(RB // DH, DH, DH)
            m = jnp.maximum(a * ins[:, :, None] + bh, 0.0) * outs[:, :, None]
            return m.reshape(RB, DH)

        m0 = mk(a0_ref, b_ref[0, :])
        m1 = mk(a1_ref, b_ref[1, :])
        out = jnp.dot(m0, w0_ref[...], preferred_element_type=jnp.float32)
        out = out + jnp.dot(m1, w1_ref[...], preferred_element_type=jnp.float32)
        out_ref[...] = out.reshape(1, RB, DH)

    return pl.pallas_call(
        body,
        grid=(NRB, NC),
        in_specs=[
            pl.BlockSpec((1, RB, DH), lambda r, c: (0, r, 0)),
            pl.BlockSpec((1, RB, DH), lambda r, c: (1, r, 0)),
            pl.BlockSpec((DH, DH), lambda r, c: (0, c)),
            pl.BlockSpec((DH, DH), lambda r, c: (1, c)),
            pl.BlockSpec((1, RB // DH, DH), lambda r, c: (1, r, 0)),
            pl.BlockSpec((1, RB // DH, DH), lambda r, c: (0, r, 0)),
            pl.BlockSpec((2, DH), lambda r, c: (0, 0)),
        ],
        out_specs=pl.BlockSpec((1, RB, DH), lambda r, c: (c, r, 0)),
        out_shape=jax.ShapeDtypeStruct((NC, NP, DH), jnp.float32),
    )(agg1, agg1, W2, W2, deg3, deg3, b2d)


def _tc_final(agg2, deg3, b2d):
    def body(a0_ref, a1_ref, din_ref, b_ref, h_ref, pool_ref, mx_sc, sm_sc):
        r = pl.program_id(0)
        ins = _rsqrt_deg(din_ref[...])

        def mk(a_ref, bh):
            a = a_ref[...].reshape(RB // DH, DH, DH)
            return (a * ins[:, :, None] + bh).reshape(RB, DH)

        hb = jnp.concatenate([mk(a0_ref, b_ref[0, :]), mk(a1_ref, b_ref[1, :])], axis=1)
        h_ref[...] = hb
        row = r * RB + lax.broadcasted_iota(jnp.int32, (RB, 1), 0)
        valid = row < N
        bmax = jnp.max(jnp.where(valid, hb, -jnp.inf), axis=0, keepdims=True)
        bsum = jnp.sum(jnp.where(valid, hb, 0.0), axis=0, keepdims=True)

        @pl.when(r == 0)
        def _init():
            mx_sc[...] = bmax
            sm_sc[...] = bsum

        @pl.when(r > 0)
        def _acc():
            mx_sc[...] = jnp.maximum(mx_sc[...], bmax)
            sm_sc[...] = sm_sc[...] + bsum

        @pl.when(r == NRB - 1)
        def _fin():
            pool_ref[...] = mx_sc[...] + sm_sc[...] * (1.0 / N)

    return pl.pallas_call(
        body,
        grid=(NRB,),
        in_specs=[
            pl.BlockSpec((1, RB, DH), lambda r: (0, r, 0)),
            pl.BlockSpec((1, RB, DH), lambda r: (1, r, 0)),
            pl.BlockSpec((1, RB // DH, DH), lambda r: (1, r, 0)),
            pl.BlockSpec((2, DH), lambda r: (0, 0)),
        ],
        out_specs=[
            pl.BlockSpec((RB, D), lambda r: (r, 0)),
            pl.BlockSpec((1, D), lambda r: (0, 0)),
        ],
        out_shape=[
            jax.ShapeDtypeStruct((NP, D), jnp.float32),
            jax.ShapeDtypeStruct((1, D), jnp.float32),
        ],
        scratch_shapes=[
            pltpu.VMEM((1, D), jnp.float32),
            pltpu.VMEM((1, D), jnp.float32),
        ],
    )(agg2, agg2, deg3, b2d)


def kernel(x, edge_index, W1, b1, W2, b2):
    src = edge_index[0].astype(jnp.int32)
    dst = edge_index[1].astype(jnp.int32)
    pad = jnp.full((EP - E,), DUMP, jnp.int32)
    srcp = jnp.concatenate([src, pad])
    dstp = jnp.concatenate([dst, pad])
    idx_deg = jnp.stack([srcp, dstp]).reshape(2, NS, EPT)
    src3 = srcp.reshape(NS, NCH, CH)
    dst3 = dstp.reshape(NS, NCH, CH)
    idx_pair = jnp.stack(
        [
            jnp.stack([src3, dst3], axis=2),
            jnp.stack([src3 + NP, dst3], axis=2),
        ]
    )
    xp = jnp.pad(x, ((0, NP - N), (0, 0)))

    deg = _sc_degrees(idx_deg)
    deg3 = deg.reshape(2, NP // DH, DH)
    h1 = _tc_h1(xp, W1, deg3)
    agg1 = _sc_message(h1.reshape(NC * NP, DH), idx_pair)
    h2 = _tc_h2(agg1, W2, deg3, b1.reshape(2, DH))
    agg2 = _sc_message(h2.reshape(NC * NP, DH), idx_pair)
    hp, pool_x = _tc_final(agg2, deg3, b2.reshape(2, DH))
    return (pool_x, hp[:N])
